# baseline (speedup 1.0000x reference)
import functools
import jax
import jax.numpy as jnp
from jax import lax
from jax.experimental import pallas as pl
from jax.experimental.pallas import tpu as pltpu
from jax.experimental.pallas import tpu_sc as plsc

B = 4096
D = 16
NW = 32
RPW = B // NW
NCH = B // 16
BIG = 1e30


def _sqrt_babylon(x):
    y = 0.5 * (x + 1.0)
    for _ in range(12):
        y = 0.5 * (y + x / y)
    return y


def _sc_body(at_hbm, pt_hbm, out_hbm, pt_v, pen_v, na2_v, apml_v):
    core = lax.axis_index("c")
    sub = lax.axis_index("s")
    wid = sub * 2 + core
    base = wid * RPW

    pltpu.sync_copy(pt_hbm, pt_v)
    zero16 = jnp.zeros((16,), jnp.float32)

    def zero_pen(c, _):
        pen_v[pl.ds(c * 16, 16)] = zero16
        return 0
    lax.fori_loop(0, NCH, zero_pen, 0)

    def col_norm(c, _):
        acc = zero16
        for d in range(D):
            v = pt_v[d, pl.ds(c * 16, 16)]
            acc = acc + v * v
        na = _sqrt_babylon(acc)
        for d in range(D):
            pt_v[d, pl.ds(c * 16, 16)] = pt_v[d, pl.ds(c * 16, 16)] / na
        return 0
    lax.fori_loop(0, NCH, col_norm, 0)

    def row_norm(g, _):
        acc = zero16
        for d in range(D):
            v = apml_v[d, pl.ds(g * 16, 16)]
            acc = acc + v * v
        na2_v[pl.ds(g * 16, 16)] = _sqrt_babylon(acc)
        return 0
    pltpu.sync_copy(at_hbm.at[:, pl.ds(base, RPW)], apml_v.at[0:16, :])
    lax.fori_loop(0, RPW // 16, row_norm, 0)

    iota16 = lax.iota(jnp.int32, 16)

    def row_loop(r, _):
        gidx = base + r
        splats = []
        for d in range(D):
            idx0 = jnp.full((16,), d, jnp.int32)
            idx1 = jnp.full((16,), r, jnp.int32)
            splats.append(plsc.load_gather(apml_v, [idx0, idx1]))
        pn_i = plsc.load_gather(
            pt_v, [iota16, jnp.full((16,), gidx, jnp.int32)])
        a_vec = plsc.load_gather(
            apml_v, [iota16, jnp.full((16,), r, jnp.int32)])
        ap_raw = lax.reduce_sum_p.bind(a_vec * pn_i, axes=(0,))

        lane0 = iota16 == 0
        gidx_v = jnp.full((16,), gidx, jnp.int32)
        plsc.store_scatter(pen_v, [gidx_v],
                           jnp.full((16,), BIG, jnp.float32), mask=lane0)

        def chunk_loop(c, m):
            acc = splats[0] * pt_v[0, pl.ds(c * 16, 16)]
            for d in range(1, D):
                acc = acc + splats[d] * pt_v[d, pl.ds(c * 16, 16)]
            acc = acc + pen_v[pl.ds(c * 16, 16)]
            return jnp.minimum(m, acc)

        m = lax.fori_loop(0, NCH, chunk_loop,
                          jnp.full((16,), jnp.inf, jnp.float32))
        plsc.store_scatter(pen_v, [gidx_v], zero16, mask=lane0)
        min_raw = lax.reduce_min_p.bind(m, axes=(0,))
        r_v = jnp.full((16,), r, jnp.int32)
        plsc.store_scatter(apml_v, [jnp.full((16,), 16, jnp.int32), r_v],
                           jnp.full((16,), ap_raw, jnp.float32), mask=lane0)
        plsc.store_scatter(apml_v, [jnp.full((16,), 17, jnp.int32), r_v],
                           jnp.full((16,), min_raw, jnp.float32), mask=lane0)
        return 0

    lax.fori_loop(0, RPW, row_loop, 0)

    def loss_loop(g, _):
        ap = apml_v[16, pl.ds(g * 16, 16)]
        mn = apml_v[17, pl.ds(g * 16, 16)]
        inv = na2_v[pl.ds(g * 16, 16)]
        loss = jnp.maximum(1.0 + (ap - mn) / inv, 0.0)
        apml_v[18, pl.ds(g * 16, 16)] = loss
        return 0
    lax.fori_loop(0, RPW // 16, loss_loop, 0)
    pltpu.sync_copy(apml_v.at[18, :], out_hbm.at[pl.ds(base, RPW)])


def kernel(anchor, positive):
    at = anchor.T
    pt = positive.T
    mesh = plsc.VectorSubcoreMesh(core_axis_name="c", subcore_axis_name="s")
    losses = pl.kernel(
        _sc_body,
        mesh=mesh,
        compiler_params=pltpu.CompilerParams(needs_layout_passes=False),
        out_type=jax.ShapeDtypeStruct((B,), jnp.float32),
        scratch_types=[
            pltpu.VMEM((D, B), jnp.float32),
            pltpu.VMEM((B,), jnp.float32),
            pltpu.VMEM((RPW,), jnp.float32),
            pltpu.VMEM((19, RPW), jnp.float32),
        ],
    )(at, pt)
    return jnp.mean(losses)


# speedup vs baseline: 21.3209x; 21.3209x over previous
import functools
import jax
import jax.numpy as jnp
from jax.experimental import pallas as pl


def _tc_body(a_ref, p_ref, out_ref, *, batch, col_chunk):
    a = a_ref[...]
    p = p_ref[...]
    a_n = a * jax.lax.rsqrt(jnp.sum(a * a, axis=1, keepdims=True))
    p_n = p * jax.lax.rsqrt(jnp.sum(p * p, axis=1, keepdims=True))

    eye = (jax.lax.broadcasted_iota(jnp.int32, (col_chunk, col_chunk), 0) ==
           jax.lax.broadcasted_iota(jnp.int32, (col_chunk, col_chunk), 1))
    chunk_mins = []
    for c in range(batch // col_chunk):
        lo = c * col_chunk
        hi = lo + col_chunk
        p_c = p_n[lo:hi, :]
        s_c = jax.lax.dot_general(a_n, p_c, (((1,), (1,)), ((), ())),
                                  preferred_element_type=jnp.float32
                                  ).astype(jnp.bfloat16)
        parts = []
        if lo > 0:
            parts.append(jnp.min(s_c[:lo, :], axis=1, keepdims=True))
        mid = jnp.where(eye, jnp.bfloat16(jnp.inf), s_c[lo:hi, :])
        parts.append(jnp.min(mid, axis=1, keepdims=True))
        if hi < batch:
            parts.append(jnp.min(s_c[hi:, :], axis=1, keepdims=True))
        chunk_mins.append(jnp.concatenate(parts, axis=0))
    an = chunk_mins[0]
    for m in chunk_mins[1:]:
        an = jnp.minimum(an, m)
    an = an.astype(jnp.float32)
    ap = jnp.sum(a_n * p_n, axis=1, keepdims=True)
    loss = jnp.sum(jnp.maximum(1.0 + ap - an, 0.0)) * (1.0 / batch)
    out_ref[...] = jnp.full(out_ref.shape, loss, jnp.float32)


def kernel(anchor, positive):
    batch, dim = anchor.shape
    out = pl.pallas_call(
        functools.partial(_tc_body, batch=batch, col_chunk=1024),
        out_shape=jax.ShapeDtypeStruct((8, 128), jnp.float32),
    )(anchor, positive)
    return out[0, 0]


# speedup vs baseline: 32.1374x; 1.5073x over previous
import functools
import jax
import jax.numpy as jnp
from jax.experimental import pallas as pl
from jax.experimental.pallas import tpu as pltpu


def _tc_body(at_ref, pt_ref, out_ref, *, batch, col_chunk):
    at = at_ref[...]
    pt = pt_ref[...]
    a_nt = at * jax.lax.rsqrt(jnp.sum(at * at, axis=0, keepdims=True))
    p_nt = pt * jax.lax.rsqrt(jnp.sum(pt * pt, axis=0, keepdims=True))

    eye = (jax.lax.broadcasted_iota(jnp.int32, (col_chunk, col_chunk), 0) ==
           jax.lax.broadcasted_iota(jnp.int32, (col_chunk, col_chunk), 1))
    chunk_mins = []
    for c in range(batch // col_chunk):
        lo = c * col_chunk
        hi = lo + col_chunk
        s_c = jax.lax.dot_general(a_nt, p_nt[:, lo:hi],
                                  (((0,), (0,)), ((), ())),
                                  preferred_element_type=jnp.float32)
        parts = []
        if lo > 0:
            parts.append(jnp.min(s_c[:lo, :], axis=1, keepdims=True))
        mid = jnp.where(eye, jnp.inf, s_c[lo:hi, :])
        parts.append(jnp.min(mid, axis=1, keepdims=True))
        if hi < batch:
            parts.append(jnp.min(s_c[hi:, :], axis=1, keepdims=True))
        chunk_mins.append(jnp.concatenate(parts, axis=0))
    an = chunk_mins[0]
    for m in chunk_mins[1:]:
        an = jnp.minimum(an, m)
    ap = jnp.sum(a_nt * p_nt, axis=0, keepdims=True)
    ap_t = jax.lax.transpose(ap, (1, 0))
    loss = jnp.sum(jnp.maximum(1.0 + ap_t - an, 0.0)) * (1.0 / batch)
    out_ref[...] = jnp.full(out_ref.shape, loss, jnp.float32)


def kernel(anchor, positive):
    batch, dim = anchor.shape
    out = pl.pallas_call(
        functools.partial(_tc_body, batch=batch, col_chunk=1024),
        out_shape=jax.ShapeDtypeStruct((8, 128), jnp.float32),
        compiler_params=pltpu.CompilerParams(
            fuse_transposed_lhs_in_matmul=True),
    )(anchor.T, positive.T)
    return out[0, 0]
